# Initial kernel scaffold; baseline (speedup 1.0000x reference)
import jax
import jax.numpy as jnp
from jax.experimental import pallas as pl


def kernel(x, edge_residue, edge_seq, edge_knn, W_res0, b_res0, W_seq0, b_seq0, W_knn0, b_knn0, W_res1, b_res1, W_seq1, b_seq1, W_knn1, b_knn1, fcW0, fcb0, fcW1, fcb1, bn_g0, bn_b0):
    raise NotImplementedError("write your pallas kernel here")


# speedup vs baseline: 8.0531x; 8.0531x over previous
import functools

import jax
import jax.numpy as jnp
from jax import lax
from jax.experimental import pallas as pl
from jax.experimental.pallas import tpu as pltpu
from jax.experimental.pallas import tpu_sc as plsc

N = 10000
E = 320000
H = 128
NC = 2
NS = 16
NW = NC * NS
CH = 80
ROWS_TOTAL = E // CH
CPW = ROWS_TOTAL // NW
STRIPE = 624
TAILN = N - NS * STRIPE
TAILOFF = NS * STRIPE
RB = 8
CPW_MAIN = 120
REM = CPW - CPW_MAIN
DGB = 5
HH = H // 2
GRID = 5
NBLK = N // GRID

_mesh = plsc.VectorSubcoreMesh(core_axis_name="c", subcore_axis_name="s")


def _sc_degrees(s0, d0, s1, d1, s2, d2, onehot, zrows8):
  @functools.partial(
      pl.kernel,
      out_type=jax.ShapeDtypeStruct((NC, N, 8), jnp.float32),
      mesh=_mesh,
      compiler_params=pltpu.CompilerParams(use_tc_tiling_on_sc=False),
      scratch_types=[
          pltpu.VMEM((CPW, CH), jnp.int32),
          pltpu.VMEM((6, CH, 8), jnp.float32),
          pltpu.VMEM_SHARED((N, 8), jnp.float32),
          pltpu.SemaphoreType.DMA((DGB,)),
      ],
  )
  def deg_kernel(s0r, d0r, s1r, d1r, s2r, d2r, oh_hbm, z8_hbm, out_hbm,
                 idx_v, oh_v, acc_sh, sem):
    c = lax.axis_index("c")
    s = lax.axis_index("s")
    wid = c * NS + s
    pltpu.sync_copy(oh_hbm, oh_v)
    pltpu.sync_copy(z8_hbm.at[pl.ds(0, STRIPE)],
                    acc_sh.at[pl.ds(s * STRIPE, STRIPE)])

    @pl.when(s == NS - 1)
    def _():
      pltpu.sync_copy(z8_hbm.at[pl.ds(0, TAILN)],
                      acc_sh.at[pl.ds(TAILOFF, TAILN)])

    plsc.subcore_barrier()
    for l, er in enumerate((s0r, d0r, s1r, d1r, s2r, d2r)):
      pltpu.sync_copy(er.at[wid], idx_v)

      for b in range(DGB):
        pltpu.async_copy(oh_v.at[l], acc_sh.at[idx_v.at[b]],
                         sem.at[b], add=True)

      @pl.loop(0, CPW - DGB, step=DGB)
      def _grp(j, _l=l):
        for b in range(DGB):
          pltpu.make_async_copy(oh_v.at[_l], acc_sh.at[idx_v.at[j + b]],
                                sem.at[b]).wait()
          pltpu.async_copy(oh_v.at[_l], acc_sh.at[idx_v.at[j + DGB + b]],
                           sem.at[b], add=True)

      for b in range(DGB):
        pltpu.make_async_copy(oh_v.at[l], acc_sh.at[idx_v.at[CPW - DGB + b]],
                              sem.at[b]).wait()

    plsc.subcore_barrier()
    pltpu.sync_copy(acc_sh.at[pl.ds(s * STRIPE, STRIPE)],
                    out_hbm.at[c, pl.ds(s * STRIPE, STRIPE)])

    @pl.when(s == NS - 1)
    def _():
      pltpu.sync_copy(acc_sh.at[pl.ds(TAILOFF, TAILN)],
                      out_hbm.at[c, pl.ds(TAILOFF, TAILN)])

  return deg_kernel(s0, d0, s1, d1, s2, d2, onehot, zrows8)


def _sc_agg(zs, s0, d0, s1, d1, s2, d2, zrows):
  @functools.partial(
      pl.kernel,
      out_type=jax.ShapeDtypeStruct((NC, 3, 2, N, HH), jnp.float32),
      mesh=_mesh,
      compiler_params=pltpu.CompilerParams(use_tc_tiling_on_sc=False),
      scratch_types=[
          pltpu.VMEM((CPW, CH), jnp.int32),
          pltpu.VMEM((CPW, CH), jnp.int32),
          pltpu.VMEM((RB, CH, HH), jnp.float32),
          pltpu.VMEM_SHARED((N, HH), jnp.float32),
          pltpu.SemaphoreType.DMA((RB,)),
          pltpu.SemaphoreType.DMA((RB,)),
      ],
  )
  def agg_kernel(z00, z01, z10, z11, z20, z21, s0r, d0r, s1r, d1r, s2r, d2r,
                 zr_hbm, out_hbm, sidx_v, didx_v, rows_v, acc_sh, gsem, ssem):
    c = lax.axis_index("c")
    s = lax.axis_index("s")
    wid = c * NS + s
    zrefs = ((z00, z01), (z10, z11), (z20, z21))
    erefs = ((s0r, d0r), (s1r, d1r), (s2r, d2r))
    for r in range(3):
      pltpu.sync_copy(erefs[r][0].at[wid], sidx_v)
      pltpu.sync_copy(erefs[r][1].at[wid], didx_v)
      for hf in range(2):
        pltpu.sync_copy(zr_hbm.at[pl.ds(0, STRIPE)],
                        acc_sh.at[pl.ds(s * STRIPE, STRIPE)])

        @pl.when(s == NS - 1)
        def _():
          pltpu.sync_copy(zr_hbm.at[pl.ds(0, TAILN)],
                          acc_sh.at[pl.ds(TAILOFF, TAILN)])

        plsc.subcore_barrier()
        zr = zrefs[r][hf]

        for b in range(REM):
          pltpu.async_copy(zr.at[sidx_v.at[CPW_MAIN + b]], rows_v.at[b],
                           gsem.at[b])
        for b in range(REM):
          pltpu.make_async_copy(zr.at[sidx_v.at[CPW_MAIN + b]], rows_v.at[b],
                                gsem.at[b]).wait()
          pltpu.async_copy(rows_v.at[b], acc_sh.at[didx_v.at[CPW_MAIN + b]],
                           ssem.at[b], add=True)
        for b in range(REM):
          pltpu.make_async_copy(rows_v.at[b],
                                acc_sh.at[didx_v.at[CPW_MAIN + b]],
                                ssem.at[b]).wait()

        for b in range(RB):
          pltpu.async_copy(zr.at[sidx_v.at[b]], rows_v.at[b], gsem.at[b])

        @pl.loop(0, CPW_MAIN - RB, step=RB)
        def _grp(g, _zr=zr):
          for b in range(RB):
            pltpu.make_async_copy(_zr.at[sidx_v.at[g + b]], rows_v.at[b],
                                  gsem.at[b]).wait()
            pltpu.async_copy(rows_v.at[b], acc_sh.at[didx_v.at[g + b]],
                             ssem.at[b], add=True)
          for b in range(RB):
            pltpu.make_async_copy(rows_v.at[b], acc_sh.at[didx_v.at[g + b]],
                                  ssem.at[b]).wait()
            pltpu.async_copy(_zr.at[sidx_v.at[g + RB + b]], rows_v.at[b],
                             gsem.at[b])

        for b in range(RB):
          pltpu.make_async_copy(zr.at[sidx_v.at[CPW_MAIN - RB + b]],
                                rows_v.at[b], gsem.at[b]).wait()
          pltpu.async_copy(rows_v.at[b],
                           acc_sh.at[didx_v.at[CPW_MAIN - RB + b]],
                           ssem.at[b], add=True)
        for b in range(RB):
          pltpu.make_async_copy(rows_v.at[b],
                                acc_sh.at[didx_v.at[CPW_MAIN - RB + b]],
                                ssem.at[b]).wait()

        plsc.subcore_barrier()
        pltpu.sync_copy(acc_sh.at[pl.ds(s * STRIPE, STRIPE)],
                        out_hbm.at[c, r, hf, pl.ds(s * STRIPE, STRIPE)])

        @pl.when(s == NS - 1)
        def _():
          pltpu.sync_copy(acc_sh.at[pl.ds(TAILOFF, TAILN)],
                          out_hbm.at[c, r, hf, pl.ds(TAILOFF, TAILN)])

        plsc.subcore_barrier()

  return agg_kernel(*zs, s0, d0, s1, d1, s2, d2, zrows)


def _dot(a, b):
  return jnp.dot(a, b, preferred_element_type=jnp.float32,
                 precision=lax.Precision.HIGHEST)


def _tc_matmul_norm(hist, x, wa, wb, wc):
  def body(h_ref, x_ref, wa_r, wb_r, wc_r, nrm, z00, z01, z10, z11, z20, z21):
    deg = h_ref[0] + h_ref[1]
    nb = lax.rsqrt(jnp.where(deg > 0., deg, 1.))
    nrm[...] = nb
    xb = x_ref[...]
    for r, (w_r, zlo, zhi) in enumerate(
        ((wa_r, z00, z01), (wb_r, z10, z11), (wc_r, z20, z21))):
      z = _dot(xb, w_r[...]) * nb[:, 2 * r:2 * r + 1]
      zlo[...] = z[:, :HH]
      zhi[...] = z[:, HH:]

  n_spec = pl.BlockSpec((NBLK, H), lambda i: (i, 0))
  h_spec = pl.BlockSpec((NBLK, HH), lambda i: (i, 0))
  w_spec = pl.BlockSpec((H, H), lambda i: (0, 0))
  return pl.pallas_call(
      body, grid=(GRID,),
      in_specs=[pl.BlockSpec((NC, NBLK, 8), lambda i: (0, i, 0)),
                n_spec, w_spec, w_spec, w_spec],
      out_specs=[pl.BlockSpec((NBLK, 8), lambda i: (i, 0))] + [h_spec] * 6,
      out_shape=[jax.ShapeDtypeStruct((N, 8), jnp.float32)] +
                [jax.ShapeDtypeStruct((N, HH), jnp.float32)] * 6,
  )(hist, x, wa, wb, wc)


def _tc_combine_fc(P, nrm, b0, b1, b2, fcw, fcb):
  def body(p, nr, b0r, b1r, b2r, w_r, fb_r, t_out, st_out):
    nb = nr[...]
    h = (jnp.concatenate([p[0, 0, 0] + p[1, 0, 0],
                          p[0, 0, 1] + p[1, 0, 1]], 1) * nb[:, 1:2] +
         jnp.concatenate([p[0, 1, 0] + p[1, 1, 0],
                          p[0, 1, 1] + p[1, 1, 1]], 1) * nb[:, 3:4] +
         jnp.concatenate([p[0, 2, 0] + p[1, 2, 0],
                          p[0, 2, 1] + p[1, 2, 1]], 1) * nb[:, 5:6])
    h = h + (b0r[...] + b1r[...] + b2r[...])
    t = jnp.maximum(_dot(h, w_r[...]) + fb_r[...], 0.)
    t_out[...] = t
    s1 = jnp.sum(t, 0, keepdims=True)
    s2 = jnp.sum(t * t, 0, keepdims=True)
    st_out[...] = jnp.concatenate(
        [s1, s2, jnp.zeros((6, H), jnp.float32)], 0)[None]

  n_spec = pl.BlockSpec((NBLK, H), lambda i: (i, 0))
  b_spec = pl.BlockSpec((1, H), lambda i: (0, 0))
  return pl.pallas_call(
      body, grid=(GRID,),
      in_specs=[pl.BlockSpec((NC, 3, 2, NBLK, HH), lambda i: (0, 0, 0, i, 0)),
                pl.BlockSpec((NBLK, 8), lambda i: (i, 0)),
                b_spec, b_spec, b_spec,
                pl.BlockSpec((H, H), lambda i: (0, 0)), b_spec],
      out_specs=[n_spec, pl.BlockSpec((1, 8, H), lambda i: (i, 0, 0))],
      out_shape=[jax.ShapeDtypeStruct((N, H), jnp.float32),
                 jax.ShapeDtypeStruct((GRID, 8, H), jnp.float32)],
  )(P, nrm, b0, b1, b2, fcw, fcb)


def _tc_bn_matmul3(t, stats, nrm, g, b, wa, wb, wc):
  def body(t_r, st_r, nr, g_r, b_r, wa_r, wb_r, wc_r,
           z00, z01, z10, z11, z20, z21):
    st = st_r[...]
    mean = jnp.sum(st[:, 0, :], 0, keepdims=True) * (1.0 / N)
    ex2 = jnp.sum(st[:, 1, :], 0, keepdims=True) * (1.0 / N)
    var = ex2 - mean * mean
    xh = (t_r[...] - mean) * lax.rsqrt(var + 1e-5) * g_r[...] + b_r[...]
    nb = nr[...]
    for r, (w_r, zlo, zhi) in enumerate(
        ((wa_r, z00, z01), (wb_r, z10, z11), (wc_r, z20, z21))):
      z = _dot(xh, w_r[...]) * nb[:, 2 * r:2 * r + 1]
      zlo[...] = z[:, :HH]
      zhi[...] = z[:, HH:]

  n_spec = pl.BlockSpec((NBLK, H), lambda i: (i, 0))
  h_spec = pl.BlockSpec((NBLK, HH), lambda i: (i, 0))
  b_spec = pl.BlockSpec((1, H), lambda i: (0, 0))
  w_spec = pl.BlockSpec((H, H), lambda i: (0, 0))
  return pl.pallas_call(
      body, grid=(GRID,),
      in_specs=[n_spec,
                pl.BlockSpec((GRID, 8, H), lambda i: (0, 0, 0)),
                pl.BlockSpec((NBLK, 8), lambda i: (i, 0)),
                b_spec, b_spec, w_spec, w_spec, w_spec],
      out_specs=[h_spec] * 6,
      out_shape=[jax.ShapeDtypeStruct((N, HH), jnp.float32)] * 6,
  )(t, stats, nrm, g, b, wa, wb, wc)


def _tc_final(P, nrm, b0, b1, b2, fcw, fcb):
  def body(p, nr, b0r, b1r, b2r, w_r, fb_r, o_ref):
    nb = nr[...]
    h = (jnp.concatenate([p[0, 0, 0] + p[1, 0, 0],
                          p[0, 0, 1] + p[1, 0, 1]], 1) * nb[:, 1:2] +
         jnp.concatenate([p[0, 1, 0] + p[1, 1, 0],
                          p[0, 1, 1] + p[1, 1, 1]], 1) * nb[:, 3:4] +
         jnp.concatenate([p[0, 2, 0] + p[1, 2, 0],
                          p[0, 2, 1] + p[1, 2, 1]], 1) * nb[:, 5:6])
    h = h + (b0r[...] + b1r[...] + b2r[...])
    o_ref[...] = _dot(h, w_r[...]) + fb_r[...]

  n_spec = pl.BlockSpec((NBLK, H), lambda i: (i, 0))
  b_spec = pl.BlockSpec((1, H), lambda i: (0, 0))
  return pl.pallas_call(
      body, grid=(GRID,),
      in_specs=[pl.BlockSpec((NC, 3, 2, NBLK, HH), lambda i: (0, 0, 0, i, 0)),
                pl.BlockSpec((NBLK, 8), lambda i: (i, 0)),
                b_spec, b_spec, b_spec,
                pl.BlockSpec((H, H), lambda i: (0, 0)), b_spec],
      out_specs=n_spec,
      out_shape=jax.ShapeDtypeStruct((N, H), jnp.float32),
  )(P, nrm, b0, b1, b2, fcw, fcb)


def kernel(x, edge_residue, edge_seq, edge_knn, W_res0, b_res0, W_seq0, b_seq0,
           W_knn0, b_knn0, W_res1, b_res1, W_seq1, b_seq1, W_knn1, b_knn1,
           fcW0, fcb0, fcW1, fcb1, bn_g0, bn_b0):
  s0 = edge_residue[0].reshape(NW, CPW, CH)
  d0 = edge_residue[1].reshape(NW, CPW, CH)
  s1 = edge_seq[0].reshape(NW, CPW, CH)
  d1 = edge_seq[1].reshape(NW, CPW, CH)
  s2 = edge_knn[0].reshape(NW, CPW, CH)
  d2 = edge_knn[1].reshape(NW, CPW, CH)

  onehot = jnp.broadcast_to(
      jnp.eye(8, dtype=jnp.float32)[:6, None, :], (6, CH, 8))
  zrows8 = jnp.zeros((STRIPE, 8), jnp.float32)
  zrows = jnp.zeros((STRIPE, HH), jnp.float32)

  b0 = b_res0.reshape(1, H)
  b1 = b_seq0.reshape(1, H)
  b2 = b_knn0.reshape(1, H)
  b0b = b_res1.reshape(1, H)
  b1b = b_seq1.reshape(1, H)
  b2b = b_knn1.reshape(1, H)

  hist = _sc_degrees(s0, d0, s1, d1, s2, d2, onehot, zrows8)
  nrm, *zs1 = _tc_matmul_norm(hist, x, W_res0, W_seq0, W_knn0)
  P1 = _sc_agg(zs1, s0, d0, s1, d1, s2, d2, zrows)
  t, stats = _tc_combine_fc(P1, nrm, b0, b1, b2, fcW0, fcb0.reshape(1, H))
  zs2 = _tc_bn_matmul3(t, stats, nrm, bn_g0.reshape(1, H),
                       bn_b0.reshape(1, H), W_res1, W_seq1, W_knn1)
  P2 = _sc_agg(zs2, s0, d0, s1, d1, s2, d2, zrows)
  return _tc_final(P2, nrm, b0b, b1b, b2b, fcW1, fcb1.reshape(1, H))
